# baseline (speedup 1.0000x reference)
import jax
import jax.numpy as jnp
from jax import lax
from jax.experimental import pallas as pl
from jax.experimental.pallas import tpu as pltpu
from jax.experimental.pallas import tpu_sc as plsc

N_NODES = 10000
NP = 10240
E = 320000
D = 128
NC = 2
NS = 16
NW = NC * NS
K = 128
EPT = E // NW
CH = 10
NCH = 8
ROWS_PER_TILE = NP // NS
CPY = 32
V = 16


def _sc_aggregate_body(x_hbm, src_hbm, dst_hbm, agg_out, deg_out,
                       sidxa, didxa, sidxb, didxb, rows_e, rows_o, zbuf,
                       dzbuf, ones, acc, dacc, gsem_e, gsem_o, ssem, dsem,
                       isema, isemb):
    cid = lax.axis_index("c")
    sid = lax.axis_index("s")
    wid = sid * NC + cid
    base = pl.multiple_of(sid * ROWS_PER_TILE, CPY)

    zero_v = jnp.zeros((V,), jnp.float32)
    one_v = jnp.ones((V,), jnp.float32)

    def zrow(r, _):
        for c in range(D // V):
            zbuf[r, pl.ds(c * V, V)] = zero_v
        return 0

    lax.fori_loop(0, CPY, zrow, 0)
    for c in range(ROWS_PER_TILE // V):
        dzbuf[pl.ds(c * V, V)] = zero_v
    for c in range(K // V):
        ones[pl.ds(c * V, V)] = one_v

    for q in range(ROWS_PER_TILE // CPY):
        r0 = base + q * CPY
        pltpu.sync_copy(zbuf, acc.at[pl.ds(r0, CPY)])
    pltpu.sync_copy(dzbuf, dacc.at[pl.ds(base, ROWS_PER_TILE)])
    plsc.subcore_barrier()

    bufs = (rows_e, rows_o)
    gsems = (gsem_e, gsem_o)

    def group(sidx, didx):
        gcp = pltpu.async_copy(x_hbm.at[sidx.at[0]], bufs[0], gsems[0])
        scp = None
        dcps = []
        for r in range(CH):
            gcp.wait()
            if scp is not None:
                scp.wait()
            if r < CH - 1:
                gcp = pltpu.async_copy(x_hbm.at[sidx.at[r + 1]],
                                       bufs[(r + 1) % 2], gsems[(r + 1) % 2])
            scp = pltpu.async_copy(bufs[r % 2], acc.at[didx.at[r]], ssem,
                                   add=True)
            dcps.append(
                pltpu.async_copy(ones, dacc.at[didx.at[r]], dsem, add=True))
        scp.wait()
        for dcp in dcps:
            dcp.wait()

    def prefetch(jj, sidx, didx, isem):
        return (pltpu.async_copy(src_hbm.at[wid, jj], sidx, isem),
                pltpu.async_copy(dst_hbm.at[wid, jj], didx, isem))

    def drain_idx(sidx, didx, isem):
        pltpu.make_async_copy(src_hbm.at[wid, 0], sidx, isem).wait()
        pltpu.make_async_copy(dst_hbm.at[wid, 0], didx, isem).wait()

    prefetch(0, sidxa, didxa, isema)
    drain_idx(sidxa, didxa, isema)
    prefetch(1, sidxb, didxb, isemb)

    def body(m, _):
        group(sidxa, didxa)
        drain_idx(sidxb, didxb, isemb)
        prefetch(lax.rem(2 * m + 2, NCH), sidxa, didxa, isema)
        group(sidxb, didxb)
        drain_idx(sidxa, didxa, isema)
        prefetch(lax.rem(2 * m + 3, NCH), sidxb, didxb, isemb)
        return 0

    lax.fori_loop(0, NCH // 2, body, 0)
    drain_idx(sidxb, didxb, isemb)
    plsc.subcore_barrier()

    for q in range(ROWS_PER_TILE // CPY):
        r0 = base + q * CPY
        pltpu.sync_copy(acc.at[pl.ds(r0, CPY)], zbuf)
        pltpu.sync_copy(zbuf, agg_out.at[cid, pl.ds(r0, CPY)])
    pltpu.sync_copy(dacc.at[pl.ds(base, ROWS_PER_TILE)], dzbuf)
    pltpu.sync_copy(dzbuf, deg_out.at[cid, 0, pl.ds(base, ROWS_PER_TILE)])


def _sc_aggregate(x, src3, dst3):
    mesh = plsc.VectorSubcoreMesh(core_axis_name="c", subcore_axis_name="s")
    f = pl.kernel(
        _sc_aggregate_body,
        out_type=[jax.ShapeDtypeStruct((NC, NP, D), jnp.float32),
                  jax.ShapeDtypeStruct((NC, 8, NP), jnp.float32)],
        mesh=mesh,
        scratch_types=[
            pltpu.VMEM((CH, K), jnp.int32),
            pltpu.VMEM((CH, K), jnp.int32),
            pltpu.VMEM((CH, K), jnp.int32),
            pltpu.VMEM((CH, K), jnp.int32),
            pltpu.VMEM((K, D), jnp.float32),
            pltpu.VMEM((K, D), jnp.float32),
            pltpu.VMEM((CPY, D), jnp.float32),
            pltpu.VMEM((ROWS_PER_TILE,), jnp.float32),
            pltpu.VMEM((K,), jnp.float32),
            pltpu.VMEM_SHARED((NP, D), jnp.float32),
            pltpu.VMEM_SHARED((NP,), jnp.float32),
            pltpu.SemaphoreType.DMA,
            pltpu.SemaphoreType.DMA,
            pltpu.SemaphoreType.DMA,
            pltpu.SemaphoreType.DMA,
            pltpu.SemaphoreType.DMA,
            pltpu.SemaphoreType.DMA,
        ],
    )
    return f(x, src3, dst3)


def _combine_body(x_ref, agg_ref, deg_ref, ws_ref, wn_ref, b_ref, o_ref):
    a = agg_ref[0] + agg_ref[1]
    d = jnp.maximum(deg_ref[0, 0] + deg_ref[1, 0], 1.0)
    h = a / d[:, None]
    o = jnp.dot(x_ref[...], ws_ref[...], preferred_element_type=jnp.float32)
    o += jnp.dot(h, wn_ref[...], preferred_element_type=jnp.float32)
    o_ref[...] = o + b_ref[...]


def _combine(xp, agg3, deg3, Ws, Wn, b):
    BR = 512
    blk = pl.BlockSpec((BR, D), lambda i: (i, 0))
    wblk = pl.BlockSpec((D, D), lambda i: (0, 0))
    return pl.pallas_call(
        _combine_body,
        grid=(NP // BR,),
        in_specs=[blk,
                  pl.BlockSpec((NC, BR, D), lambda i: (0, i, 0)),
                  pl.BlockSpec((NC, 8, BR), lambda i: (0, 0, i)),
                  wblk, wblk,
                  pl.BlockSpec((1, D), lambda i: (0, 0))],
        out_specs=blk,
        out_shape=jax.ShapeDtypeStruct((NP, D), jnp.float32),
    )(xp, agg3, deg3, Ws, Wn, b.reshape(1, D))


def kernel(inputs, edge_index, W_self1, W_neigh1, b1, W_self2, W_neigh2, b2):
    eidx = edge_index.astype(jnp.int32)
    pad_e = ((0, 0), (0, NCH * CH * K - EPT))
    src3 = jnp.pad(eidx[0].reshape(NW, EPT), pad_e).reshape(NW, NCH, CH, K)
    dst3 = jnp.pad(eidx[1].reshape(NW, EPT), pad_e,
                   constant_values=NP - 1).reshape(NW, NCH, CH, K)
    xp = jnp.pad(inputs, ((0, NP - N_NODES), (0, 0)))

    agg1, deg1 = _sc_aggregate(xp, src3, dst3)
    x1 = _combine(xp, agg1, deg1, W_self1, W_neigh1, b1)
    agg2, _ = _sc_aggregate(x1, src3, dst3)
    x2 = _combine(x1, agg2, deg1, W_self2, W_neigh2, b2)
    return x2[:N_NODES]


# speedup vs baseline: 1.0022x; 1.0022x over previous
import jax
import jax.numpy as jnp
from jax import lax
from jax.experimental import pallas as pl
from jax.experimental.pallas import tpu as pltpu
from jax.experimental.pallas import tpu_sc as plsc

N_NODES = 10000
NP = 10240
E = 320000
D = 128
NC = 2
NS = 16
NW = NC * NS
K = 128
EPT = E // NW
CH = 8
NCH = 10
ROWS_PER_TILE = NP // NS
CPY = 64
V = 16


def _sc_aggregate_body(x_hbm, src_hbm, dst_hbm, agg_out, deg_out,
                       sidxa, didxa, sidxb, didxb, rows_e, rows_o, zbuf,
                       dzbuf, ones, acc, dacc, gsem_e, gsem_o, ssem, dsem,
                       isema, isemb):
    cid = lax.axis_index("c")
    sid = lax.axis_index("s")
    wid = sid * NC + cid
    base = pl.multiple_of(sid * ROWS_PER_TILE, CPY)

    zero_v = jnp.zeros((V,), jnp.float32)
    one_v = jnp.ones((V,), jnp.float32)

    def zrow(r, _):
        for c in range(D // V):
            zbuf[r, pl.ds(c * V, V)] = zero_v
        return 0

    lax.fori_loop(0, CPY, zrow, 0)
    for c in range(ROWS_PER_TILE // V):
        dzbuf[pl.ds(c * V, V)] = zero_v
    for c in range(K // V):
        ones[pl.ds(c * V, V)] = one_v

    for q in range(ROWS_PER_TILE // CPY):
        r0 = base + q * CPY
        pltpu.sync_copy(zbuf, acc.at[pl.ds(r0, CPY)])
    pltpu.sync_copy(dzbuf, dacc.at[pl.ds(base, ROWS_PER_TILE)])
    plsc.subcore_barrier()

    bufs = (rows_e, rows_o)
    gsems = (gsem_e, gsem_o)

    def group(sidx, didx):
        gcp = pltpu.async_copy(x_hbm.at[sidx.at[0]], bufs[0], gsems[0])
        scp = None
        dcps = []
        for r in range(CH):
            gcp.wait()
            if scp is not None:
                scp.wait()
            if r < CH - 1:
                gcp = pltpu.async_copy(x_hbm.at[sidx.at[r + 1]],
                                       bufs[(r + 1) % 2], gsems[(r + 1) % 2])
            scp = pltpu.async_copy(bufs[r % 2], acc.at[didx.at[r]], ssem,
                                   add=True)
            dcps.append(
                pltpu.async_copy(ones, dacc.at[didx.at[r]], dsem, add=True))
        scp.wait()
        for dcp in dcps:
            dcp.wait()

    def prefetch(jj, sidx, didx, isem):
        return (pltpu.async_copy(src_hbm.at[wid, jj], sidx, isem),
                pltpu.async_copy(dst_hbm.at[wid, jj], didx, isem))

    def drain_idx(sidx, didx, isem):
        pltpu.make_async_copy(src_hbm.at[wid, 0], sidx, isem).wait()
        pltpu.make_async_copy(dst_hbm.at[wid, 0], didx, isem).wait()

    prefetch(0, sidxa, didxa, isema)
    drain_idx(sidxa, didxa, isema)
    prefetch(1, sidxb, didxb, isemb)

    def body(m, _):
        group(sidxa, didxa)
        drain_idx(sidxb, didxb, isemb)
        prefetch(lax.rem(2 * m + 2, NCH), sidxa, didxa, isema)
        group(sidxb, didxb)
        drain_idx(sidxa, didxa, isema)
        prefetch(lax.rem(2 * m + 3, NCH), sidxb, didxb, isemb)
        return 0

    lax.fori_loop(0, NCH // 2, body, 0)
    drain_idx(sidxb, didxb, isemb)
    plsc.subcore_barrier()

    for q in range(ROWS_PER_TILE // CPY):
        r0 = base + q * CPY
        pltpu.sync_copy(acc.at[pl.ds(r0, CPY)], zbuf)
        pltpu.sync_copy(zbuf, agg_out.at[cid, pl.ds(r0, CPY)])
    pltpu.sync_copy(dacc.at[pl.ds(base, ROWS_PER_TILE)], dzbuf)
    pltpu.sync_copy(dzbuf, deg_out.at[cid, 0, pl.ds(base, ROWS_PER_TILE)])


def _sc_aggregate(x, src3, dst3):
    mesh = plsc.VectorSubcoreMesh(core_axis_name="c", subcore_axis_name="s")
    f = pl.kernel(
        _sc_aggregate_body,
        out_type=[jax.ShapeDtypeStruct((NC, NP, D), jnp.float32),
                  jax.ShapeDtypeStruct((NC, 8, NP), jnp.float32)],
        mesh=mesh,
        scratch_types=[
            pltpu.VMEM((CH, K), jnp.int32),
            pltpu.VMEM((CH, K), jnp.int32),
            pltpu.VMEM((CH, K), jnp.int32),
            pltpu.VMEM((CH, K), jnp.int32),
            pltpu.VMEM((K, D), jnp.float32),
            pltpu.VMEM((K, D), jnp.float32),
            pltpu.VMEM((CPY, D), jnp.float32),
            pltpu.VMEM((ROWS_PER_TILE,), jnp.float32),
            pltpu.VMEM((K,), jnp.float32),
            pltpu.VMEM_SHARED((NP, D), jnp.float32),
            pltpu.VMEM_SHARED((NP,), jnp.float32),
            pltpu.SemaphoreType.DMA,
            pltpu.SemaphoreType.DMA,
            pltpu.SemaphoreType.DMA,
            pltpu.SemaphoreType.DMA,
            pltpu.SemaphoreType.DMA,
            pltpu.SemaphoreType.DMA,
        ],
    )
    return f(x, src3, dst3)


def _combine_body(x_ref, agg_ref, deg_ref, ws_ref, wn_ref, b_ref, o_ref):
    a = agg_ref[0] + agg_ref[1]
    d = jnp.maximum(deg_ref[0, 0] + deg_ref[1, 0], 1.0)
    h = a / d[:, None]
    o = jnp.dot(x_ref[...], ws_ref[...], preferred_element_type=jnp.float32)
    o += jnp.dot(h, wn_ref[...], preferred_element_type=jnp.float32)
    o_ref[...] = o + b_ref[...]


def _combine(xp, agg3, deg3, Ws, Wn, b):
    BR = 512
    blk = pl.BlockSpec((BR, D), lambda i: (i, 0))
    wblk = pl.BlockSpec((D, D), lambda i: (0, 0))
    return pl.pallas_call(
        _combine_body,
        grid=(NP // BR,),
        in_specs=[blk,
                  pl.BlockSpec((NC, BR, D), lambda i: (0, i, 0)),
                  pl.BlockSpec((NC, 8, BR), lambda i: (0, 0, i)),
                  wblk, wblk,
                  pl.BlockSpec((1, D), lambda i: (0, 0))],
        out_specs=blk,
        out_shape=jax.ShapeDtypeStruct((NP, D), jnp.float32),
    )(xp, agg3, deg3, Ws, Wn, b.reshape(1, D))


def kernel(inputs, edge_index, W_self1, W_neigh1, b1, W_self2, W_neigh2, b2):
    eidx = edge_index.astype(jnp.int32)
    pad_e = ((0, 0), (0, NCH * CH * K - EPT))
    src3 = jnp.pad(eidx[0].reshape(NW, EPT), pad_e).reshape(NW, NCH, CH, K)
    dst3 = jnp.pad(eidx[1].reshape(NW, EPT), pad_e,
                   constant_values=NP - 1).reshape(NW, NCH, CH, K)
    xp = jnp.pad(inputs, ((0, NP - N_NODES), (0, 0)))

    agg1, deg1 = _sc_aggregate(xp, src3, dst3)
    x1 = _combine(xp, agg1, deg1, W_self1, W_neigh1, b1)
    agg2, _ = _sc_aggregate(x1, src3, dst3)
    x2 = _combine(x1, agg2, deg1, W_self2, W_neigh2, b2)
    return x2[:N_NODES]
